# baseline (speedup 1.0000x reference)
import functools
import math

import jax
import jax.numpy as jnp
from jax import lax
from jax.experimental import pallas as pl
from jax.experimental.pallas import tpu as pltpu
from jax.experimental.pallas import tpu_sc as plsc

_LV_LR = 0.01
_SIGMA = 1.0

_NC = 2
_NS = 16
_L = 16


def _make_to_rows(N, D, BLK, interpret=False):
    def body(src_ref, dst_ref):
        t = src_ref[...].T
        dst_ref[...] = jnp.concatenate(
            [t, jnp.zeros((BLK, 128 - D), jnp.float32)], axis=1)

    return pl.pallas_call(
        body,
        grid=(pl.cdiv(N, BLK),),
        in_specs=[pl.BlockSpec((D, BLK), lambda i: (0, i))],
        out_specs=pl.BlockSpec((BLK, 128), lambda i: (i, 0)),
        out_shape=jax.ShapeDtypeStruct((N, 128), jnp.float32),
        interpret=interpret,
    )


def _make_to_cols(N, D, BLK, interpret=False):
    def body(src_ref, dst_ref):
        dst_ref[...] = src_ref[:, 0:D].T

    return pl.pallas_call(
        body,
        grid=(pl.cdiv(N, BLK),),
        in_specs=[pl.BlockSpec((BLK, 128), lambda i: (i, 0))],
        out_specs=pl.BlockSpec((D, BLK), lambda i: (0, i)),
        out_shape=jax.ShapeDtypeStruct((D, N), jnp.float32),
        interpret=interpret,
    )


def _make_gather(N, D, B, interpret=False):
    NW = _NC * _NS
    bpw = B // NW
    nch = bpw // 128
    mesh = plsc.VectorSubcoreMesh(
        core_axis_name="c", subcore_axis_name="s",
        num_cores=_NC, num_subcores=_NS,
    )

    @functools.partial(
        pl.kernel,
        mesh=mesh,
        out_type=jax.ShapeDtypeStruct((B, D), jnp.float32),
        scratch_types=[
            pltpu.VMEM((nch, 128), jnp.int32),
            pltpu.VMEM((bpw, 128), jnp.float32),
            pltpu.VMEM((bpw, D), jnp.float32),
            pltpu.SemaphoreType.DMA,
        ],
        compiler_params=pltpu.CompilerParams(
            use_tc_tiling_on_sc=False, needs_layout_passes=False),
        interpret=interpret,
    )
    def gk(tab_hbm, idx_hbm, lv_hbm, idx_v, rows_v, lv_v, sem):
        wid = lax.axis_index("s") * _NC + lax.axis_index("c")
        pltpu.sync_copy(idx_hbm.at[pl.ds(wid * nch, nch)], idx_v)
        cps = [
            pltpu.async_copy(
                tab_hbm.at[idx_v.at[c]], rows_v.at[pl.ds(c * 128, 128)], sem
            )
            for c in range(nch)
        ]
        for cp in cps:
            cp.wait()

        def compact(i, _):
            lv_v[i, 0:16] = rows_v[i, 0:16]
            lv_v[i, 16:32] = rows_v[i, 16:32]
            return 0

        lax.fori_loop(0, bpw, compact, 0)
        pltpu.sync_copy(lv_v, lv_hbm.at[pl.ds(wid * bpw, bpw)])

    return gk


def _mlp_body(B, X, lv_ref, datat_ref, noise_ref, w1_ref, b1_ref, w2_ref,
              b2_ref, upd_ref, loss_ref):
    lv = lv_ref[...]
    h_pre = jnp.dot(lv, w1_ref[...], preferred_element_type=jnp.float32)
    h_pre = h_pre + b1_ref[...]
    h = jnp.maximum(h_pre, 0.0)
    pred_t = lax.dot_general(w2_ref[...], h, (((0,), (1,)), ((), ())),
                             preferred_element_type=jnp.float32)
    raw_t = datat_ref[...] - (pred_t + b2_ref[...])
    resid_t = raw_t * (1.0 / (_SIGMA ** 2))
    g_h = lax.dot_general(resid_t, w2_ref[...], (((0,), (1,)), ((), ())),
                          preferred_element_type=jnp.float32)
    g_h = jnp.where(h_pre > 0, g_h, 0.0)
    g_lv = lax.dot_general(g_h, w1_ref[...], (((1,), (1,)), ((), ())),
                           preferred_element_type=jnp.float32)
    upd_ref[...] = (_LV_LR * (g_lv - lv)
                    + math.sqrt(2.0 * _LV_LR) * noise_ref[...])

    part = (0.5 * jnp.sum(lv * lv)
            + (0.5 / (_SIGMA ** 2)) * jnp.sum(raw_t * raw_t))

    @pl.when(pl.program_id(0) == 0)
    def _():
        loss_ref[...] = jnp.zeros_like(loss_ref)

    loss_ref[...] += part.reshape(1, 1)

    D = lv.shape[1]
    const = 0.5 * (D * math.log(2.0 * math.pi)
                   + X * math.log(2.0 * math.pi * _SIGMA ** 2))

    @pl.when(pl.program_id(0) == pl.num_programs(0) - 1)
    def _():
        loss_ref[...] = loss_ref[...] / B + jnp.float32(const)


def _make_mlp(B, D, H, X, BB, interpret=False):
    nb = B // BB
    return pl.pallas_call(
        functools.partial(_mlp_body, B, X),
        grid=(nb,),
        in_specs=[
            pl.BlockSpec((BB, D), lambda i: (i, 0)),
            pl.BlockSpec((X, BB), lambda i: (0, i)),
            pl.BlockSpec((BB, D), lambda i: (i, 0)),
            pl.BlockSpec((D, H), lambda i: (0, 0)),
            pl.BlockSpec((1, H), lambda i: (0, 0)),
            pl.BlockSpec((H, X), lambda i: (0, 0)),
            pl.BlockSpec((X, 1), lambda i: (0, 0)),
        ],
        out_specs=[
            pl.BlockSpec((BB, D), lambda i: (i, 0)),
            pl.BlockSpec((1, 1), lambda i: (0, 0)),
        ],
        out_shape=[
            jax.ShapeDtypeStruct((B, D), jnp.float32),
            jax.ShapeDtypeStruct((1, 1), jnp.float32),
        ],
        interpret=interpret,
    )


def _make_scatter(N, D, B, interpret=False):
    upt = B // _NS
    nch = upt // 128
    mesh = plsc.VectorSubcoreMesh(
        core_axis_name="c", subcore_axis_name="s",
        num_cores=1, num_subcores=_NS,
    )

    @functools.partial(
        pl.kernel,
        mesh=mesh,
        out_type=jax.ShapeDtypeStruct((N, 16), jnp.int32),
        scratch_types=[
            pltpu.VMEM((nch, 128), jnp.int32),
            pltpu.VMEM((nch, 128), jnp.int32),
            pltpu.VMEM((upt, 16), jnp.int32),
            pltpu.VMEM((upt, D), jnp.float32),
            pltpu.VMEM((128, D), jnp.float32),
            pltpu.VMEM((128, 128), jnp.float32),
            pltpu.VMEM_SHARED((B, D), jnp.float32),
        ],
        compiler_params=pltpu.CompilerParams(
            use_tc_tiling_on_sc=False, needs_layout_passes=False),
        interpret=interpret,
    )
    def sk(out_ref, lv_hbm, idx_hbm, upd_hbm, tmp_hbm,
           idx_v, win_v, w_mat, acc_v, lv_v, row_v, acc_sh):
        sid = lax.axis_index("s")
        base = sid * upt

        pltpu.sync_copy(idx_hbm.at[pl.ds(sid * nch, nch)], idx_v)

        def initj(i, _):
            w_mat[i, 0:16] = jnp.broadcast_to(base + i, (16,)).astype(jnp.int32)
            return 0

        lax.fori_loop(0, upt, initj, 0)

        def zrow(i, _):
            acc_v[i, 0:16] = jnp.zeros((_L,), jnp.float32)
            acc_v[i, 16:32] = jnp.zeros((_L,), jnp.float32)
            return 0

        lax.fori_loop(0, upt, zrow, 0)

        def zpad(i, _):
            for k in range(8):
                row_v[i, pl.ds(k * 16, 16)] = jnp.zeros((_L,), jnp.float32)
            return 0

        lax.fori_loop(0, 128, zpad, 0)

        for c in range(nch):
            pltpu.sync_copy(w_mat.at[pl.ds(c * 128, 128)],
                            tmp_hbm.at[idx_v.at[c]])
        pltpu.sync_copy(acc_v, acc_sh.at[pl.ds(base, upt)])
        plsc.subcore_barrier()

        for c in range(nch):
            pltpu.sync_copy(tmp_hbm.at[idx_v.at[c]],
                            w_mat.at[pl.ds(c * 128, 128)])
        lanes = lax.iota(jnp.int32, _L)
        zero16 = jnp.zeros((_L,), jnp.int32)
        for g in range(upt // _L):
            vals = plsc.load_gather(w_mat, [lanes + g * _L, zero16])
            win_v[g // 8, pl.ds((g % 8) * _L, _L)] = vals

        for c in range(nch):
            pltpu.sync_copy(upd_hbm.at[pl.ds(base + c * 128, 128)], lv_v)
            pltpu.sync_copy(lv_v, acc_sh.at[win_v.at[c]], add=True)
        plsc.subcore_barrier()

        for c in range(nch):
            pltpu.sync_copy(acc_sh.at[win_v.at[c]],
                            acc_v.at[pl.ds(c * 128, 128)])
            pltpu.sync_copy(lv_hbm.at[pl.ds(base + c * 128, 128)], lv_v)

            def addrow(i, _, c=c):
                row_v[i, 0:16] = (lv_v[i, 0:16] + acc_v[c * 128 + i, 0:16])
                row_v[i, 16:32] = (lv_v[i, 16:32] + acc_v[c * 128 + i, 16:32])
                return 0

            lax.fori_loop(0, 128, addrow, 0)
            pltpu.sync_copy(row_v, out_ref.at[idx_v.at[c]])

    return sk


def _run(mem, d_idx, data_batch, W1, b1, W2, b2, noise, interpret=False):
    N, D = mem.shape
    B = d_idx.shape[0]
    H = W1.shape[1]
    X = W2.shape[1]

    idx2d = d_idx.reshape(B // 128, 128)

    mem128 = _make_to_rows(N, D, 8192, interpret)(mem.T)
    lv = _make_gather(N, D, B, interpret)(mem128, idx2d)

    upd, loss2d = _make_mlp(B, D, H, X, 1024, interpret)(
        lv, data_batch.T, noise, W1, b1.reshape(1, H), W2, b2.reshape(X, 1)
    )

    mem128b, lv2 = lax.optimization_barrier((mem128, lv))
    out_ref = jax.new_ref(mem128b)
    _make_scatter(N, D, B, interpret)(out_ref, lv2, idx2d, upd)
    new_mem = _make_to_cols(N, D, 8192, interpret)(out_ref[...]).T
    return loss2d[0, 0], new_mem


def kernel(mem, d_idx, data_batch, W1, b1, W2, b2, noise):
    return _run(mem, d_idx, data_batch, W1, b1, W2, b2, noise)


# speedup vs baseline: 1.1308x; 1.1308x over previous
import functools
import math

import jax
import jax.numpy as jnp
from jax import lax
from jax.experimental import pallas as pl
from jax.experimental.pallas import tpu as pltpu
from jax.experimental.pallas import tpu_sc as plsc

_LV_LR = 0.01
_SIGMA = 1.0

_NC = 2
_NS = 16
_L = 16


def _make_to_rows(N, D, BLK, interpret=False):
    def body(src_ref, dst_ref):
        t = src_ref[...].T
        dst_ref[...] = jnp.concatenate(
            [t, jnp.zeros((BLK, 128 - D), jnp.float32)], axis=1)

    return pl.pallas_call(
        body,
        grid=(pl.cdiv(N, BLK),),
        in_specs=[pl.BlockSpec((D, BLK), lambda i: (0, i))],
        out_specs=pl.BlockSpec((BLK, 128), lambda i: (i, 0)),
        out_shape=jax.ShapeDtypeStruct((N, 128), jnp.float32),
        interpret=interpret,
    )


def _make_to_cols(N, D, BLK, interpret=False):
    def body(src_ref, dst_ref):
        dst_ref[...] = src_ref[:, 0:D].T

    return pl.pallas_call(
        body,
        grid=(pl.cdiv(N, BLK),),
        in_specs=[pl.BlockSpec((BLK, 128), lambda i: (i, 0))],
        out_specs=pl.BlockSpec((D, BLK), lambda i: (0, i)),
        out_shape=jax.ShapeDtypeStruct((D, N), jnp.float32),
        interpret=interpret,
    )


def _make_gather(N, D, B, interpret=False):
    NW = _NC * _NS
    bpw = B // NW
    nch = bpw // 128
    mesh = plsc.VectorSubcoreMesh(
        core_axis_name="c", subcore_axis_name="s",
        num_cores=_NC, num_subcores=_NS,
    )

    @functools.partial(
        pl.kernel,
        mesh=mesh,
        out_type=jax.ShapeDtypeStruct((B, D), jnp.float32),
        scratch_types=[
            pltpu.VMEM((nch, 128), jnp.int32),
            pltpu.VMEM((bpw, 128), jnp.float32),
            pltpu.VMEM((bpw, D), jnp.float32),
            pltpu.SemaphoreType.DMA,
        ],
        compiler_params=pltpu.CompilerParams(
            use_tc_tiling_on_sc=False, needs_layout_passes=False),
        interpret=interpret,
    )
    def gk(tab_hbm, idx_hbm, lv_hbm, idx_v, rows_v, lv_v, sem):
        wid = lax.axis_index("s") * _NC + lax.axis_index("c")
        pltpu.sync_copy(idx_hbm.at[pl.ds(wid * nch, nch)], idx_v)
        cps = [
            pltpu.async_copy(
                tab_hbm.at[idx_v.at[c]], rows_v.at[pl.ds(c * 128, 128)], sem
            )
            for c in range(nch)
        ]
        for cp in cps:
            cp.wait()

        def compact(i, _):
            lv_v[i, 0:16] = rows_v[i, 0:16]
            lv_v[i, 16:32] = rows_v[i, 16:32]
            return 0

        lax.fori_loop(0, bpw, compact, 0)
        pltpu.sync_copy(lv_v, lv_hbm.at[pl.ds(wid * bpw, bpw)])

    return gk


def _mlp_body(B, X, lv_ref, datat_ref, noise_ref, w1_ref, b1_ref, w2_ref,
              b2_ref, upd_ref, loss_ref):
    lv = lv_ref[...]
    h_pre = jnp.dot(lv, w1_ref[...], preferred_element_type=jnp.float32)
    h_pre = h_pre + b1_ref[...]
    h = jnp.maximum(h_pre, 0.0)
    pred_t = lax.dot_general(w2_ref[...], h, (((0,), (1,)), ((), ())),
                             preferred_element_type=jnp.float32)
    raw_t = datat_ref[...] - (pred_t + b2_ref[...])
    resid_t = raw_t * (1.0 / (_SIGMA ** 2))
    g_h = lax.dot_general(resid_t, w2_ref[...], (((0,), (1,)), ((), ())),
                          preferred_element_type=jnp.float32)
    g_h = jnp.where(h_pre > 0, g_h, 0.0)
    g_lv = lax.dot_general(g_h, w1_ref[...], (((1,), (1,)), ((), ())),
                           preferred_element_type=jnp.float32)
    upd_ref[...] = (_LV_LR * (g_lv - lv)
                    + math.sqrt(2.0 * _LV_LR) * noise_ref[...])

    part = (0.5 * jnp.sum(lv * lv)
            + (0.5 / (_SIGMA ** 2)) * jnp.sum(raw_t * raw_t))

    @pl.when(pl.program_id(0) == 0)
    def _():
        loss_ref[...] = jnp.zeros_like(loss_ref)

    loss_ref[...] += part.reshape(1, 1)

    D = lv.shape[1]
    const = 0.5 * (D * math.log(2.0 * math.pi)
                   + X * math.log(2.0 * math.pi * _SIGMA ** 2))

    @pl.when(pl.program_id(0) == pl.num_programs(0) - 1)
    def _():
        loss_ref[...] = loss_ref[...] / B + jnp.float32(const)


def _make_mlp(B, D, H, X, BB, interpret=False):
    nb = B // BB
    return pl.pallas_call(
        functools.partial(_mlp_body, B, X),
        grid=(nb,),
        in_specs=[
            pl.BlockSpec((BB, D), lambda i: (i, 0)),
            pl.BlockSpec((X, BB), lambda i: (0, i)),
            pl.BlockSpec((BB, D), lambda i: (i, 0)),
            pl.BlockSpec((D, H), lambda i: (0, 0)),
            pl.BlockSpec((1, H), lambda i: (0, 0)),
            pl.BlockSpec((H, X), lambda i: (0, 0)),
            pl.BlockSpec((X, 1), lambda i: (0, 0)),
        ],
        out_specs=[
            pl.BlockSpec((BB, D), lambda i: (i, 0)),
            pl.BlockSpec((1, 1), lambda i: (0, 0)),
        ],
        out_shape=[
            jax.ShapeDtypeStruct((B, D), jnp.float32),
            jax.ShapeDtypeStruct((1, 1), jnp.float32),
        ],
        interpret=interpret,
    )


def _make_scatter(N, D, B, interpret=False):
    upt = B // _NS
    nch = upt // 128
    mesh = plsc.VectorSubcoreMesh(
        core_axis_name="c", subcore_axis_name="s",
        num_cores=1, num_subcores=_NS,
    )

    @functools.partial(
        pl.kernel,
        mesh=mesh,
        out_type=jax.ShapeDtypeStruct((N, 16), jnp.int32),
        scratch_types=[
            pltpu.VMEM((nch, 128), jnp.int32),
            pltpu.VMEM((nch, 128), jnp.int32),
            pltpu.VMEM((upt, 16), jnp.int32),
            pltpu.VMEM((upt, D), jnp.float32),
            pltpu.VMEM((128, D), jnp.float32),
            pltpu.VMEM((128, 128), jnp.float32),
            pltpu.VMEM_SHARED((B, D), jnp.float32),
        ],
        compiler_params=pltpu.CompilerParams(
            use_tc_tiling_on_sc=False, needs_layout_passes=False),
        interpret=interpret,
    )
    def sk(out_ref, lv_hbm, idx_hbm, upd_hbm, tmp_hbm,
           idx_v, win_v, w_mat, acc_v, lv_v, row_v, acc_sh):
        sid = lax.axis_index("s")
        base = sid * upt

        pltpu.sync_copy(idx_hbm.at[pl.ds(sid * nch, nch)], idx_v)

        def initj(i, _):
            w_mat[i, 0:16] = jnp.broadcast_to(base + i, (16,)).astype(jnp.int32)
            return 0

        lax.fori_loop(0, upt, initj, 0)

        def zrow(i, _):
            acc_v[i, 0:16] = jnp.zeros((_L,), jnp.float32)
            acc_v[i, 16:32] = jnp.zeros((_L,), jnp.float32)
            return 0

        lax.fori_loop(0, upt, zrow, 0)

        def zpad(i, _):
            for k in range(8):
                row_v[i, pl.ds(k * 16, 16)] = jnp.zeros((_L,), jnp.float32)
            return 0

        lax.fori_loop(0, 128, zpad, 0)

        for c in range(nch):
            pltpu.sync_copy(w_mat.at[pl.ds(c * 128, 128)],
                            tmp_hbm.at[idx_v.at[c]])
        pltpu.sync_copy(acc_v, acc_sh.at[pl.ds(base, upt)])
        plsc.subcore_barrier()

        for c in range(nch):
            pltpu.sync_copy(tmp_hbm.at[idx_v.at[c]],
                            w_mat.at[pl.ds(c * 128, 128)])
        lanes = lax.iota(jnp.int32, _L)
        zero16 = jnp.zeros((_L,), jnp.int32)
        for g in range(upt // _L):
            vals = plsc.load_gather(w_mat, [lanes + g * _L, zero16])
            win_v[g // 8, pl.ds((g % 8) * _L, _L)] = vals

        for c in range(nch):
            pltpu.sync_copy(upd_hbm.at[pl.ds(base + c * 128, 128)], lv_v)
            pltpu.sync_copy(lv_v, acc_sh.at[win_v.at[c]], add=True)
        plsc.subcore_barrier()

        for c in range(nch):
            pltpu.sync_copy(acc_sh.at[win_v.at[c]],
                            acc_v.at[pl.ds(c * 128, 128)])
            pltpu.sync_copy(lv_hbm.at[pl.ds(base + c * 128, 128)], lv_v)

            def addrow(i, _, c=c):
                row_v[i, 0:16] = (lv_v[i, 0:16] + acc_v[c * 128 + i, 0:16])
                row_v[i, 16:32] = (lv_v[i, 16:32] + acc_v[c * 128 + i, 16:32])
                return 0

            lax.fori_loop(0, 128, addrow, 0)
            pltpu.sync_copy(row_v, out_ref.at[idx_v.at[c]])

    return sk


def _run(mem, d_idx, data_batch, W1, b1, W2, b2, noise, interpret=False):
    N, D = mem.shape
    B = d_idx.shape[0]
    H = W1.shape[1]
    X = W2.shape[1]

    idx2d = d_idx.reshape(B // 128, 128)

    mem128 = _make_to_rows(N, D, 16384, interpret)(mem.T)
    lv = _make_gather(N, D, B, interpret)(mem128, idx2d)

    upd, loss2d = _make_mlp(B, D, H, X, 1024, interpret)(
        lv, data_batch.T, noise, W1, b1.reshape(1, H), W2, b2.reshape(X, 1)
    )

    out_ref = jax.new_ref(mem128)
    _make_scatter(N, D, B, interpret)(out_ref, lv, idx2d, upd)
    new_mem = _make_to_cols(N, D, 16384, interpret)(out_ref[...]).T
    return loss2d[0, 0], new_mem


def kernel(mem, d_idx, data_batch, W1, b1, W2, b2, noise):
    return _run(mem, d_idx, data_batch, W1, b1, W2, b2, noise)


# speedup vs baseline: 1.1552x; 1.0216x over previous
import functools
import math

import jax
import jax.numpy as jnp
from jax import lax
from jax.experimental import pallas as pl
from jax.experimental.pallas import tpu as pltpu
from jax.experimental.pallas import tpu_sc as plsc

_LV_LR = 0.01
_SIGMA = 1.0

_NC = 2
_NS = 16
_L = 16


def _make_to_rows(N, D, BLK, interpret=False):
    def body(src_ref, dst_ref):
        t = src_ref[...].T
        dst_ref[...] = jnp.concatenate(
            [t, jnp.zeros((BLK, 128 - D), jnp.float32)], axis=1)

    return pl.pallas_call(
        body,
        grid=(pl.cdiv(N, BLK),),
        in_specs=[pl.BlockSpec((D, BLK), lambda i: (0, i))],
        out_specs=pl.BlockSpec((BLK, 128), lambda i: (i, 0)),
        out_shape=jax.ShapeDtypeStruct((N, 128), jnp.float32),
        interpret=interpret,
    )


def _make_to_cols(N, D, BLK, interpret=False):
    def body(src_ref, dst_ref):
        dst_ref[...] = src_ref[:, 0:D].T

    return pl.pallas_call(
        body,
        grid=(pl.cdiv(N, BLK),),
        in_specs=[pl.BlockSpec((BLK, 128), lambda i: (i, 0))],
        out_specs=pl.BlockSpec((D, BLK), lambda i: (0, i)),
        out_shape=jax.ShapeDtypeStruct((D, N), jnp.float32),
        interpret=interpret,
    )


def _make_gather(N, D, B, interpret=False):
    NW = _NC * _NS
    bpw = B // NW
    nch = bpw // 128
    mesh = plsc.VectorSubcoreMesh(
        core_axis_name="c", subcore_axis_name="s",
        num_cores=_NC, num_subcores=_NS,
    )

    @functools.partial(
        pl.kernel,
        mesh=mesh,
        out_type=jax.ShapeDtypeStruct((B, D), jnp.float32),
        scratch_types=[
            pltpu.VMEM((nch, 128), jnp.int32),
            pltpu.VMEM((bpw, 128), jnp.float32),
            pltpu.VMEM((bpw, D), jnp.float32),
            pltpu.SemaphoreType.DMA,
        ],
        compiler_params=pltpu.CompilerParams(
            use_tc_tiling_on_sc=False, needs_layout_passes=False),
        interpret=interpret,
    )
    def gk(tab_hbm, idx_hbm, lv_hbm, idx_v, rows_v, lv_v, sem):
        wid = lax.axis_index("s") * _NC + lax.axis_index("c")
        pltpu.sync_copy(idx_hbm.at[pl.ds(wid * nch, nch)], idx_v)
        cps = [
            pltpu.async_copy(
                tab_hbm.at[idx_v.at[c]], rows_v.at[pl.ds(c * 128, 128)], sem
            )
            for c in range(nch)
        ]
        for cp in cps:
            cp.wait()

        def compact(i, _):
            lv_v[i, 0:16] = rows_v[i, 0:16]
            lv_v[i, 16:32] = rows_v[i, 16:32]
            return 0

        lax.fori_loop(0, bpw, compact, 0)
        pltpu.sync_copy(lv_v, lv_hbm.at[pl.ds(wid * bpw, bpw)])

    return gk


def _mlp_body(B, X, lv_ref, datat_ref, noise_ref, w1_ref, b1_ref, w2_ref,
              b2_ref, upd_ref, loss_ref):
    lv = lv_ref[...]
    h_pre = jnp.dot(lv, w1_ref[...], preferred_element_type=jnp.float32)
    h_pre = h_pre + b1_ref[...]
    h = jnp.maximum(h_pre, 0.0)
    pred_t = lax.dot_general(w2_ref[...], h, (((0,), (1,)), ((), ())),
                             preferred_element_type=jnp.float32)
    raw_t = datat_ref[...] - (pred_t + b2_ref[...])
    resid_t = raw_t * (1.0 / (_SIGMA ** 2))
    g_h = lax.dot_general(resid_t, w2_ref[...], (((0,), (1,)), ((), ())),
                          preferred_element_type=jnp.float32)
    g_h = jnp.where(h_pre > 0, g_h, 0.0)
    g_lv = lax.dot_general(g_h, w1_ref[...], (((1,), (1,)), ((), ())),
                           preferred_element_type=jnp.float32)
    upd_ref[...] = (_LV_LR * (g_lv - lv)
                    + math.sqrt(2.0 * _LV_LR) * noise_ref[...])

    part = (0.5 * jnp.sum(lv * lv)
            + (0.5 / (_SIGMA ** 2)) * jnp.sum(raw_t * raw_t))

    @pl.when(pl.program_id(0) == 0)
    def _():
        loss_ref[...] = jnp.zeros_like(loss_ref)

    loss_ref[...] += part.reshape(1, 1)

    D = lv.shape[1]
    const = 0.5 * (D * math.log(2.0 * math.pi)
                   + X * math.log(2.0 * math.pi * _SIGMA ** 2))

    @pl.when(pl.program_id(0) == pl.num_programs(0) - 1)
    def _():
        loss_ref[...] = loss_ref[...] / B + jnp.float32(const)


def _make_mlp(B, D, H, X, BB, interpret=False):
    nb = B // BB
    return pl.pallas_call(
        functools.partial(_mlp_body, B, X),
        grid=(nb,),
        in_specs=[
            pl.BlockSpec((BB, D), lambda i: (i, 0)),
            pl.BlockSpec((X, BB), lambda i: (0, i)),
            pl.BlockSpec((BB, D), lambda i: (i, 0)),
            pl.BlockSpec((D, H), lambda i: (0, 0)),
            pl.BlockSpec((1, H), lambda i: (0, 0)),
            pl.BlockSpec((H, X), lambda i: (0, 0)),
            pl.BlockSpec((X, 1), lambda i: (0, 0)),
        ],
        out_specs=[
            pl.BlockSpec((BB, D), lambda i: (i, 0)),
            pl.BlockSpec((1, 1), lambda i: (0, 0)),
        ],
        out_shape=[
            jax.ShapeDtypeStruct((B, D), jnp.float32),
            jax.ShapeDtypeStruct((1, 1), jnp.float32),
        ],
        interpret=interpret,
    )


def _make_scatter(N, D, B, interpret=False):
    upt = B // _NS
    nch = upt // 128
    mesh = plsc.VectorSubcoreMesh(
        core_axis_name="c", subcore_axis_name="s",
        num_cores=1, num_subcores=_NS,
    )

    @functools.partial(
        pl.kernel,
        mesh=mesh,
        out_type=jax.ShapeDtypeStruct((N, 16), jnp.int32),
        scratch_types=[
            pltpu.VMEM((nch, 128), jnp.int32),
            pltpu.VMEM((nch, 128), jnp.int32),
            pltpu.VMEM((upt, 16), jnp.int32),
            pltpu.VMEM((upt, D), jnp.float32),
            pltpu.VMEM((128, D), jnp.float32),
            pltpu.VMEM((128, 128), jnp.float32),
            pltpu.VMEM_SHARED((B, D), jnp.float32),
        ],
        compiler_params=pltpu.CompilerParams(
            use_tc_tiling_on_sc=False, needs_layout_passes=False),
        interpret=interpret,
    )
    def sk(out_ref, lv_hbm, idx_hbm, upd_hbm, tmp_hbm,
           idx_v, win_v, w_mat, acc_v, lv_v, row_v, acc_sh):
        sid = lax.axis_index("s")
        base = sid * upt

        pltpu.sync_copy(idx_hbm.at[pl.ds(sid * nch, nch)], idx_v)

        def initj(i, _):
            w_mat[i, 0:16] = jnp.broadcast_to(base + i, (16,)).astype(jnp.int32)
            return 0

        lax.fori_loop(0, upt, initj, 0)

        def zrow(i, _):
            acc_v[i, 0:16] = jnp.zeros((_L,), jnp.float32)
            acc_v[i, 16:32] = jnp.zeros((_L,), jnp.float32)
            return 0

        lax.fori_loop(0, upt, zrow, 0)

        def zpad(i, _):
            for k in range(8):
                row_v[i, pl.ds(k * 16, 16)] = jnp.zeros((_L,), jnp.float32)
            return 0

        lax.fori_loop(0, 128, zpad, 0)

        for c in range(nch):
            pltpu.sync_copy(w_mat.at[pl.ds(c * 128, 128)],
                            tmp_hbm.at[idx_v.at[c]])
        pltpu.sync_copy(acc_v, acc_sh.at[pl.ds(base, upt)])
        plsc.subcore_barrier()

        for c in range(nch):
            pltpu.sync_copy(tmp_hbm.at[idx_v.at[c]],
                            w_mat.at[pl.ds(c * 128, 128)])
        lanes = lax.iota(jnp.int32, _L)
        zero16 = jnp.zeros((_L,), jnp.int32)
        for g in range(upt // _L):
            vals = plsc.load_gather(w_mat, [lanes + g * _L, zero16])
            win_v[g // 8, pl.ds((g % 8) * _L, _L)] = vals

        for c in range(nch):
            pltpu.sync_copy(upd_hbm.at[pl.ds(base + c * 128, 128)], lv_v)
            pltpu.sync_copy(lv_v, acc_sh.at[win_v.at[c]], add=True)
        plsc.subcore_barrier()

        for c in range(nch):
            pltpu.sync_copy(acc_sh.at[win_v.at[c]],
                            acc_v.at[pl.ds(c * 128, 128)])
            pltpu.sync_copy(lv_hbm.at[pl.ds(base + c * 128, 128)], lv_v)

            def addrow(i, _, c=c):
                row_v[i, 0:16] = (lv_v[i, 0:16] + acc_v[c * 128 + i, 0:16])
                row_v[i, 16:32] = (lv_v[i, 16:32] + acc_v[c * 128 + i, 16:32])
                return 0

            lax.fori_loop(0, 128, addrow, 0)
            pltpu.sync_copy(row_v, out_ref.at[idx_v.at[c]])

    return sk


def _run(mem, d_idx, data_batch, W1, b1, W2, b2, noise, interpret=False):
    N, D = mem.shape
    B = d_idx.shape[0]
    H = W1.shape[1]
    X = W2.shape[1]

    idx2d = d_idx.reshape(B // 128, 128)

    mem128 = _make_to_rows(N, D, 32768, interpret)(mem.T)
    lv = _make_gather(N, D, B, interpret)(mem128, idx2d)

    upd, loss2d = _make_mlp(B, D, H, X, 1024, interpret)(
        lv, data_batch.T, noise, W1, b1.reshape(1, H), W2, b2.reshape(X, 1)
    )

    out_ref = jax.new_ref(mem128)
    _make_scatter(N, D, B, interpret)(out_ref, lv, idx2d, upd)
    new_mem = _make_to_cols(N, D, 32768, interpret)(out_ref[...]).T
    return loss2d[0, 0], new_mem


def kernel(mem, d_idx, data_batch, W1, b1, W2, b2, noise):
    return _run(mem, d_idx, data_batch, W1, b1, W2, b2, noise)
